# baseline (speedup 1.0000x reference)
import functools

import jax
import jax.numpy as jnp
from jax import lax
from jax.experimental import pallas as pl
from jax.experimental.pallas import tpu as pltpu
from jax.experimental.pallas import tpu_sc as plsc

B = 4
S = 4096
HID = 1024
N = B * S
NC = 2
NS = 16
NW = NC * NS
T = N // NW
C = 32
NCHUNK = T // C


def _emb_body(ids_hbm, mask_hbm, table_hbm, out_hbm,
              idx_v, mask_v, buf0, buf1, g0, g1, o0, o1, msem):
    wid = lax.axis_index("s") * NC + lax.axis_index("c")
    base = wid * T

    bufs = (buf0, buf1)
    gsems = (g0, g1)
    osems = (o0, o1)

    pltpu.sync_copy(ids_hbm.at[pl.ds(base, T)], idx_v)
    pltpu.sync_copy(mask_hbm.at[pl.ds(base, T)], mask_v)
    mh = pltpu.async_copy(
        mask_v, out_hbm.at[pl.ds(base, T), pl.ds(HID, 1)], msem
    )

    def start_gather(k):
        b = k & 1
        return pltpu.async_copy(
            table_hbm.at[idx_v.at[pl.ds(k * C, C)]],
            bufs[b],
            gsems[b],
        )

    gh = {0: start_gather(0)}
    oh = {}
    for k in range(NCHUNK):
        b = k & 1
        if k + 1 < NCHUNK:
            nb = (k + 1) & 1
            if k + 1 >= 2:
                oh[nb].wait()
            gh[nb] = start_gather(k + 1)
        gh[b].wait()
        oh[b] = pltpu.async_copy(
            bufs[b],
            out_hbm.at[pl.ds(base + k * C, C), pl.ds(0, HID)],
            osems[b],
        )
    oh[(NCHUNK - 2) & 1].wait()
    oh[(NCHUNK - 1) & 1].wait()
    mh.wait()


_emb_call = functools.partial(
    pl.kernel,
    mesh=plsc.VectorSubcoreMesh(core_axis_name="c", subcore_axis_name="s"),
    out_type=jax.ShapeDtypeStruct((N, HID + 1), jnp.float32),
    compiler_params=pltpu.CompilerParams(use_tc_tiling_on_sc=False),
    scratch_types=[
        pltpu.VMEM((T,), jnp.int32),
        pltpu.VMEM((T, 1), jnp.float32),
        pltpu.VMEM((C, HID), jnp.float32),
        pltpu.VMEM((C, HID), jnp.float32),
        pltpu.SemaphoreType.DMA,
        pltpu.SemaphoreType.DMA,
        pltpu.SemaphoreType.DMA,
        pltpu.SemaphoreType.DMA,
        pltpu.SemaphoreType.DMA,
    ],
)(_emb_body)


@jax.jit
def kernel(input_ids, attention_mask, table):
    ids = input_ids.reshape(N).astype(jnp.int32)
    maskf = attention_mask.reshape(N, 1).astype(jnp.float32)
    out = _emb_call(ids, maskf, table)
    return out.reshape(B, S, HID + 1)
